# Initial kernel scaffold; baseline (speedup 1.0000x reference)
import jax
import jax.numpy as jnp
from jax.experimental import pallas as pl


def kernel(node_feats, edge_feats, pos, edge_index, Wq, Wk, Wv, Wskip, R1, b1, lng, lnb, R2, b2, ng, nb, Wout):
    raise NotImplementedError("write your pallas kernel here")


# speedup vs baseline: 15.2130x; 15.2130x over previous
import functools

import jax
import jax.numpy as jnp
from jax import lax
from jax.experimental import pallas as pl
from jax.experimental.pallas import tpu as pltpu
from jax.experimental.pallas import tpu_sc as plsc

NC = 2
NS = 16
LANES = 16

N = 10000
E = 320000
D = 128
DE = 16
H = 2
HD = D // H
RH = 32

NPAD = 10240
CB = 80
EPC = E // NS
CPC = EPC // CB
EVR = NPAD // 2
SR = NPAD // 8
EVPS = EVR // NS
SPS = SR // NS

EPW32 = E // 32

_HI = jax.lax.Precision.HIGHEST


def _mesh():
    return plsc.VectorSubcoreMesh(core_axis_name="c", subcore_axis_name="s")


_SC_PARAMS = pltpu.CompilerParams(needs_layout_passes=False,
                                  use_tc_tiling_on_sc=False)


def _dist_body(px_hbm, py_hbm, pz_hbm, src_hbm, dst_hbm, d2_hbm,
               px_v, py_v, pz_v, src_v, dst_v, d2_v):
    wid = lax.axis_index("s") * NC + lax.axis_index("c")
    base = pl.multiple_of(wid * EPW32, 8)
    pltpu.sync_copy(px_hbm, px_v)
    pltpu.sync_copy(py_hbm, py_v)
    pltpu.sync_copy(pz_hbm, pz_v)
    pltpu.sync_copy(src_hbm.at[pl.ds(base, EPW32)], src_v)
    pltpu.sync_copy(dst_hbm.at[pl.ds(base, EPW32)], dst_v)

    def body(g, carry):
        si = src_v[pl.ds(g * LANES, LANES)]
        di = dst_v[pl.ds(g * LANES, LANES)]
        dx = plsc.load_gather(px_v, [di]) - plsc.load_gather(px_v, [si])
        dy = plsc.load_gather(py_v, [di]) - plsc.load_gather(py_v, [si])
        dz = plsc.load_gather(pz_v, [di]) - plsc.load_gather(pz_v, [si])
        d2_v[pl.ds(g * LANES, LANES)] = dx * dx + dy * dy + dz * dz
        return carry

    lax.fori_loop(0, EPW32 // LANES, body, 0)
    pltpu.sync_copy(d2_v, d2_hbm.at[pl.ds(base, EPW32)])


def _sc_dist(px, py, pz, src, dst):
    f = functools.partial(
        pl.kernel,
        out_type=jax.ShapeDtypeStruct((E,), jnp.float32),
        mesh=_mesh(),
        compiler_params=_SC_PARAMS,
        scratch_types=[
            pltpu.VMEM((N,), jnp.float32),
            pltpu.VMEM((N,), jnp.float32),
            pltpu.VMEM((N,), jnp.float32),
            pltpu.VMEM((EPW32,), jnp.int32),
            pltpu.VMEM((EPW32,), jnp.int32),
            pltpu.VMEM((EPW32,), jnp.float32),
        ],
    )(_dist_body)
    return f(px, py, pz, src, dst)


def _edge_body(xq_hbm, xk_hbm, xv_hbm, rad_hbm, src_hbm, dst_hbm, zer_hbm,
               uev_hbm, us_hbm, srcb, dstb, dstbp, dst2b, dst8b,
               qv, kv, vv, rv, wv, svb, uev, us, sq, sk, sv, sr):
    cid = lax.axis_index("c")
    sid = lax.axis_index("s")

    evb = sid * EVPS
    ssb = sid * SPS
    pltpu.sync_copy(zer_hbm.at[pl.ds(evb, EVPS)], uev.at[pl.ds(evb, EVPS)])
    pltpu.sync_copy(zer_hbm.at[pl.ds(ssb, SPS)], us.at[pl.ds(ssb, SPS)])
    plsc.subcore_barrier()

    lane = lax.broadcasted_iota(jnp.int32, (LANES,), 0)
    zv = jnp.zeros((LANES,), jnp.float32)
    noff = cid * N

    def ebody(i, carry):
        h = jnp.zeros((LANES,), jnp.float32)
        vrs = []
        for j in range(4):
            dsj = pl.ds(j * LANES, LANES)
            rj = rv[i, dsj]
            h = h + qv[i, dsj] * (kv[i, dsj] * rj)
            vrs.append(vv[i, dsj] * rj)
        l = jnp.minimum(jnp.sum(h) * 0.125, 75.0)
        ev = jnp.exp(jnp.full((LANES,), l, jnp.float32))
        d = dstbp[pl.ds(i, LANES)][0]
        offw = (d & 1) * HD
        offz = HD - offw
        for j in range(4):
            wv[i, pl.ds(offw + j * LANES, LANES)] = vrs[j] * ev
            wv[i, pl.ds(offz + j * LANES, LANES)] = zv
        for j in range(8):
            svb[i, pl.ds(j * LANES, LANES)] = zv
        offs = (d & 7) * LANES
        svb[i, pl.ds(offs, LANES)] = jnp.where(lane == 0, ev, 0.0)
        return carry

    def cbody(t, carry):
        base = pl.multiple_of(sid * EPC + t * CB, 8)
        pltpu.sync_copy(src_hbm.at[pl.ds(base, CB)], srcb)
        pltpu.sync_copy(dst_hbm.at[pl.ds(base, CB)], dstb)
        pltpu.sync_copy(dst_hbm.at[pl.ds(base, CB)], dstbp.at[pl.ds(0, CB)])
        for g in range(CB // LANES):
            dsg = pl.ds(g * LANES, LANES)
            dg = dstb[dsg]
            srcb[dsg] = srcb[dsg] + noff
            dstb[dsg] = dg + noff
            dst2b[dsg] = lax.shift_right_logical(dg, 1)
            dst8b[dsg] = lax.shift_right_logical(dg, 3)
        cq = pltpu.async_copy(xq_hbm.at[dstb], qv, sq)
        ck = pltpu.async_copy(xk_hbm.at[srcb], kv, sk)
        cv = pltpu.async_copy(xv_hbm.at[srcb], vv, sv)
        rbase = pl.multiple_of(cid * E + base, 8)
        cr = pltpu.async_copy(rad_hbm.at[pl.ds(rbase, CB)], rv, sr)
        cq.wait()
        ck.wait()
        cv.wait()
        cr.wait()
        lax.fori_loop(0, CB, ebody, 0)
        pltpu.sync_copy(wv, uev.at[dst2b], add=True)
        pltpu.sync_copy(svb, us.at[dst8b], add=True)
        return carry

    lax.fori_loop(0, CPC, cbody, 0)
    plsc.subcore_barrier()
    pltpu.sync_copy(uev.at[pl.ds(evb, EVPS)],
                    uev_hbm.at[cid, pl.ds(evb, EVPS)])
    pltpu.sync_copy(us.at[pl.ds(ssb, SPS)],
                    us_hbm.at[cid, pl.ds(ssb, SPS)])


def _sc_edge(xq, xk, xv, rad, src, dst, zer):
    f = functools.partial(
        pl.kernel,
        out_type=(jax.ShapeDtypeStruct((NC, EVR, D), jnp.float32),
                  jax.ShapeDtypeStruct((NC, SR, D), jnp.float32)),
        mesh=_mesh(),
        compiler_params=_SC_PARAMS,
        scratch_types=[
            pltpu.VMEM((CB,), jnp.int32),
            pltpu.VMEM((CB,), jnp.int32),
            pltpu.VMEM((CB + LANES,), jnp.int32),
            pltpu.VMEM((CB,), jnp.int32),
            pltpu.VMEM((CB,), jnp.int32),
            pltpu.VMEM((CB, HD), jnp.float32),
            pltpu.VMEM((CB, HD), jnp.float32),
            pltpu.VMEM((CB, HD), jnp.float32),
            pltpu.VMEM((CB, HD), jnp.float32),
            pltpu.VMEM((CB, D), jnp.float32),
            pltpu.VMEM((CB, D), jnp.float32),
            pltpu.VMEM_SHARED((EVR, D), jnp.float32),
            pltpu.VMEM_SHARED((SR, D), jnp.float32),
            pltpu.SemaphoreType.DMA,
            pltpu.SemaphoreType.DMA,
            pltpu.SemaphoreType.DMA,
            pltpu.SemaphoreType.DMA,
        ],
    )(_edge_body)
    return f(xq, xk, xv, rad, src, dst, zer)


NB = 1000
EB = 2000


def _proj_body(x_ref, wq_ref, wk_ref, wv_ref, ws_ref,
               xq_ref, xk_ref, xv_ref, xs_ref):
    x = x_ref[...]
    xq = jnp.dot(x, wq_ref[...], precision=_HI)
    xk = jnp.dot(x, wk_ref[...], precision=_HI)
    xv = jnp.dot(x, wv_ref[...], precision=_HI)
    xq_ref[0] = xq[:, :HD]
    xq_ref[1] = xq[:, HD:]
    xk_ref[0] = xk[:, :HD]
    xk_ref[1] = xk[:, HD:]
    xv_ref[0] = xv[:, :HD]
    xv_ref[1] = xv[:, HD:]
    xs_ref[...] = jnp.dot(x, ws_ref[...], precision=_HI)


def _tc_proj(x, wq, wk, wv, ws):
    wspec = pl.BlockSpec((D, D), lambda i: (0, 0))
    nspec = pl.BlockSpec((NB, D), lambda i: (i, 0))
    hspec = pl.BlockSpec((NC, NB, HD), lambda i: (0, i, 0))
    hshape = jax.ShapeDtypeStruct((NC, N, HD), jnp.float32)
    xq, xk, xv, xs = pl.pallas_call(
        _proj_body,
        grid=(N // NB,),
        in_specs=[nspec, wspec, wspec, wspec, wspec],
        out_specs=[hspec, hspec, hspec, nspec],
        out_shape=[hshape, hshape, hshape,
                   jax.ShapeDtypeStruct((N, D), jnp.float32)],
    )(x, wq, wk, wv, ws)
    return (xq.reshape(NC * N, HD), xk.reshape(NC * N, HD),
            xv.reshape(NC * N, HD), xs)


def _radial_body(d2_ref, ef_ref, wd0_ref, we0_ref, b10_ref, g0_ref, be0_ref,
                 r20_ref, b20_ref, wd1_ref, we1_ref, b11_ref, g1_ref,
                 be1_ref, r21_ref, b21_ref, rad0_ref, rad1_ref):
    dist = jnp.sqrt(d2_ref[...] + 1e-12)
    ef = ef_ref[...]
    params = ((wd0_ref, we0_ref, b10_ref, g0_ref, be0_ref, r20_ref, b20_ref,
               rad0_ref),
              (wd1_ref, we1_ref, b11_ref, g1_ref, be1_ref, r21_ref, b21_ref,
               rad1_ref))
    for wd, we, b1, g, be, r2, b2, out in params:
        h = dist * wd[...] + jnp.dot(ef, we[...], precision=_HI) + b1[...]
        mu = jnp.mean(h, axis=1, keepdims=True)
        hc = h - mu
        var = jnp.mean(hc * hc, axis=1, keepdims=True)
        hn = hc / jnp.sqrt(var + 1e-5) * g[...] + be[...]
        hr = jnp.maximum(hn, 0.0)
        rad = jnp.dot(hr, r2[...], precision=_HI) + b2[...]
        out[0] = rad[:, :HD]
        out[1] = rad[:, HD:]


def _tc_radial(d2c, ef, R1, b1, lng, lnb, R2, b2):
    espec = pl.BlockSpec((EB, DE), lambda i: (i, 0))
    dspec = pl.BlockSpec((EB, 1), lambda i: (i, 0))
    ospec = pl.BlockSpec((NC, EB, HD), lambda i: (0, i, 0))

    def w(shape):
        return pl.BlockSpec(shape, lambda i: (0, 0))

    args = [d2c, ef]
    in_specs = [dspec, espec]
    for l in range(2):
        args += [R1[l, 0:1, :], R1[l, 1:, :], b1[l:l + 1, :],
                 lng[l:l + 1, :], lnb[l:l + 1, :], R2[l], b2[l:l + 1, :]]
        in_specs += [w((1, RH)), w((DE, RH)), w((1, RH)), w((1, RH)),
                     w((1, RH)), w((RH, D)), w((1, D))]
    rad0, rad1 = pl.pallas_call(
        _radial_body,
        grid=(E // EB,),
        in_specs=in_specs,
        out_specs=[ospec, ospec],
        out_shape=[jax.ShapeDtypeStruct((NC, E, HD), jnp.float32)] * 2,
    )(*args)
    return rad0.reshape(NC * E, HD), rad1.reshape(NC * E, HD)


def _combine(uev, sv, xs):
    agg_halves = []
    for c in range(NC):
        s = sv[c][:, 0:1]
        agg_halves.append(uev[c] / (jnp.broadcast_to(s, (NB, HD)) + 1e-9))
    agg = jnp.concatenate(agg_halves, axis=1)
    return agg + xs


def _norm_se3(out, g, b):
    nrm = jnp.abs(out)
    phase = out / (nrm + 1e-8)
    return jnp.maximum(nrm * g + b, 0.0) * phase


def _combine_mid_body(u_ref, s_ref, xs_ref, g_ref, b_ref, xn_ref):
    out = _combine(u_ref[...], s_ref[...], xs_ref[...])
    xn_ref[...] = _norm_se3(out, g_ref[...], b_ref[...])


def _combine_final_body(u_ref, s_ref, xs_ref, g_ref, b_ref, wout_ref, y_ref):
    out = _combine(u_ref[...], s_ref[...], xs_ref[...])
    xn = _norm_se3(out, g_ref[...], b_ref[...])
    y_ref[...] = jnp.dot(xn, wout_ref[...], precision=_HI)


def _tc_combine(uev, sv, xs, g, b, wout=None):
    uspec = pl.BlockSpec((NC, NB, HD), lambda i: (0, i, 0))
    sspec = pl.BlockSpec((NC, NB, 16), lambda i: (0, i, 0))
    nspec = pl.BlockSpec((NB, D), lambda i: (i, 0))
    gspec = pl.BlockSpec((1, D), lambda i: (0, 0))
    out_shape = jax.ShapeDtypeStruct((N, D), jnp.float32)
    if wout is None:
        return pl.pallas_call(
            _combine_mid_body,
            grid=(N // NB,),
            in_specs=[uspec, sspec, nspec, gspec, gspec],
            out_specs=nspec,
            out_shape=out_shape,
        )(uev, sv, xs, g, b)
    return pl.pallas_call(
        _combine_final_body,
        grid=(N // NB,),
        in_specs=[uspec, sspec, nspec, gspec, gspec,
                  pl.BlockSpec((D, D), lambda i: (0, 0))],
        out_specs=nspec,
        out_shape=out_shape,
    )(uev, sv, xs, g, b, wout)


def kernel(node_feats, edge_feats, pos, edge_index, Wq, Wk, Wv, Wskip,
           R1, b1, lng, lnb, R2, b2, ng, nb, Wout):
    x = node_feats[..., 0]
    ef = edge_feats[..., 0]
    src = edge_index[0]
    dst = edge_index[1]
    px = jnp.asarray(pos[:, 0])
    py = jnp.asarray(pos[:, 1])
    pz = jnp.asarray(pos[:, 2])

    d2 = _sc_dist(px, py, pz, src, dst)
    rad0, rad1 = _tc_radial(d2.reshape(E, 1), ef, R1, b1, lng, lnb, R2, b2)
    zer = jnp.zeros((EVR, D), jnp.float32)

    xq0, xk0, xv0, xs0 = _tc_proj(x, Wq[0], Wk[0], Wv[0], Wskip[0])
    uev0, us0 = _sc_edge(xq0, xk0, xv0, rad0, src, dst, zer)
    x1 = _tc_combine(uev0.reshape(NC, NPAD, HD), us0.reshape(NC, NPAD, 16),
                     xs0, ng[0:1, :], nb[0:1, :])

    xq1, xk1, xv1, xs1 = _tc_proj(x1, Wq[1], Wk[1], Wv[1], Wskip[1])
    uev1, us1 = _sc_edge(xq1, xk1, xv1, rad1, src, dst, zer)
    y = _tc_combine(uev1.reshape(NC, NPAD, HD), us1.reshape(NC, NPAD, 16),
                    xs1, ng[1:2, :], nb[1:2, :], Wout)
    return y[..., None]


# speedup vs baseline: 19.6591x; 1.2923x over previous
import functools

import jax
import jax.numpy as jnp
from jax import lax
from jax.experimental import pallas as pl
from jax.experimental.pallas import tpu as pltpu
from jax.experimental.pallas import tpu_sc as plsc

NC = 2
NS = 16
LANES = 16

N = 10000
E = 320000
D = 128
DE = 16
H = 2
HD = D // H
RH = 32

NPAD = 10240
CB = 80
EPC = E // NS
CPC = EPC // CB
SR16 = NPAD // 16

EPW32 = E // 32

_HI = jax.lax.Precision.HIGHEST


def _mesh():
    return plsc.VectorSubcoreMesh(core_axis_name="c", subcore_axis_name="s")


_SC_PARAMS = pltpu.CompilerParams(needs_layout_passes=False,
                                  use_tc_tiling_on_sc=False)


def _dist_body(px_hbm, py_hbm, pz_hbm, src_hbm, dst_hbm, d2_hbm,
               px_v, py_v, pz_v, src_v, dst_v, d2_v):
    wid = lax.axis_index("s") * NC + lax.axis_index("c")
    base = pl.multiple_of(wid * EPW32, 8)
    pltpu.sync_copy(px_hbm, px_v)
    pltpu.sync_copy(py_hbm, py_v)
    pltpu.sync_copy(pz_hbm, pz_v)
    pltpu.sync_copy(src_hbm.at[pl.ds(base, EPW32)], src_v)
    pltpu.sync_copy(dst_hbm.at[pl.ds(base, EPW32)], dst_v)

    def body(g, carry):
        si = src_v[pl.ds(g * LANES, LANES)]
        di = dst_v[pl.ds(g * LANES, LANES)]
        dx = plsc.load_gather(px_v, [di]) - plsc.load_gather(px_v, [si])
        dy = plsc.load_gather(py_v, [di]) - plsc.load_gather(py_v, [si])
        dz = plsc.load_gather(pz_v, [di]) - plsc.load_gather(pz_v, [si])
        d2_v[pl.ds(g * LANES, LANES)] = dx * dx + dy * dy + dz * dz
        return carry

    lax.fori_loop(0, EPW32 // LANES, body, 0)
    pltpu.sync_copy(d2_v, d2_hbm.at[pl.ds(base, EPW32)])


def _sc_dist(px, py, pz, src, dst):
    f = functools.partial(
        pl.kernel,
        out_type=jax.ShapeDtypeStruct((E,), jnp.float32),
        mesh=_mesh(),
        compiler_params=_SC_PARAMS,
        scratch_types=[
            pltpu.VMEM((N,), jnp.float32),
            pltpu.VMEM((N,), jnp.float32),
            pltpu.VMEM((N,), jnp.float32),
            pltpu.VMEM((EPW32,), jnp.int32),
            pltpu.VMEM((EPW32,), jnp.int32),
            pltpu.VMEM((EPW32,), jnp.float32),
        ],
    )(_dist_body)
    return f(px, py, pz, src, dst)


def _edge_body(xq_hbm, xk_hbm, xv_hbm, rad_hbm, srcx_hbm, dst_hbm,
               zer_hbm, zer2_hbm, uev_hbm, us_hbm,
               dstall,
               sga, dga, dsa, d16a, qva, kva, vva, rva, wva, sva,
               sgb, dgb, dsb, d16b, qvb, kvb, vvb, rvb, wvb, svb,
               uev, us, sema, semb, semsga, semsgb):
    cid = lax.axis_index("c")
    sid = lax.axis_index("s")

    rb = sid * (NPAD // NS)
    nps = NPAD // NS
    rb2 = sid * (SR16 // NS)
    nps2 = SR16 // NS
    pltpu.sync_copy(zer_hbm.at[pl.ds(rb, nps)], uev.at[pl.ds(rb, nps)])
    pltpu.sync_copy(zer2_hbm.at[pl.ds(rb2, nps2)], us.at[pl.ds(rb2, nps2)])

    ebase = pl.multiple_of(sid * EPC, 8)
    pltpu.sync_copy(dst_hbm.at[pl.ds(ebase, EPC)], dstall.at[pl.ds(0, EPC)])
    plsc.subcore_barrier()

    lane = lax.broadcasted_iota(jnp.int32, (LANES,), 0)
    noff = cid * N

    bufs_a = (sga, dga, dsa, d16a, qva, kva, vva, rva, wva, sva, sema,
              semsga)
    bufs_b = (sgb, dgb, dsb, d16b, qvb, kvb, vvb, rvb, wvb, svb, semb,
              semsgb)

    def issue_sg(t, bufs):
        sg, sem = bufs[0], bufs[11]
        t = jnp.minimum(t, CPC - 1)
        base = pl.multiple_of(sid * EPC, 8) + t * CB
        pltpu.async_copy(srcx_hbm.at[cid, pl.ds(base, CB)], sg, sem)

    def wait_sg(bufs):
        sg, sem = bufs[0], bufs[11]
        pltpu.make_async_copy(srcx_hbm.at[cid, pl.ds(0, CB)], sg, sem).wait()

    def issue_gathers(t, bufs):
        sg, dg, dsc, dsc16, qv, kv, vv, rv, wv, sv, sem, semsg = bufs
        t = jnp.minimum(t, CPC - 1)
        off = t * CB
        for g in range(CB // LANES):
            dsg = pl.ds(g * LANES, LANES)
            d16 = dstall[pl.ds(off + g * LANES, LANES)]
            dg[dsg] = d16 + noff
            dsc[dsg] = d16
            dsc16[dsg] = lax.shift_right_logical(d16, 4)
        wait_sg(bufs)
        pltpu.async_copy(xq_hbm.at[dg], qv, sem)
        pltpu.async_copy(xk_hbm.at[sg], kv, sem)
        pltpu.async_copy(xv_hbm.at[sg], vv, sem)
        rbase = pl.multiple_of(cid * E + sid * EPC, 8) + t * CB
        pltpu.async_copy(rad_hbm.at[pl.ds(rbase, CB)], rv, sem)

    def drain(bufs):
        sg, dg, dsc, dsc16, qv, kv, vv, rv, wv, sv, sem, semsg = bufs
        pltpu.make_async_copy(xq_hbm.at[dg], qv, sem).wait()
        pltpu.make_async_copy(xk_hbm.at[sg], kv, sem).wait()
        pltpu.make_async_copy(xv_hbm.at[sg], vv, sem).wait()
        pltpu.make_async_copy(rad_hbm.at[pl.ds(0, CB)], rv, sem).wait()

    def work(t, bufs):
        sg, dg, dsc, dsc16, qv, kv, vv, rv, wv, sv, sem, semsg = bufs
        off = jnp.minimum(t, CPC - 1) * CB

        def ebody(i, carry):
            h = jnp.zeros((LANES,), jnp.float32)
            vrs = []
            for j in range(4):
                dsj = pl.ds(j * LANES, LANES)
                rj = rv[i, dsj]
                h = h + qv[i, dsj] * (kv[i, dsj] * rj)
                vrs.append(vv[i, dsj] * rj)
            l = jnp.minimum(jnp.sum(h) * 0.125, 75.0)
            ev = jnp.exp(jnp.full((LANES,), l, jnp.float32))
            for j in range(4):
                wv[i, pl.ds(j * LANES, LANES)] = vrs[j] * ev
            d = dstall[pl.ds(off + i, LANES)][0]
            sv[i, :] = jnp.where(lane == (d & 15), ev, 0.0)
            return carry

        lax.fori_loop(0, CB, ebody, 0, unroll=2)
        pltpu.sync_copy(wv, uev.at[dsc], add=True)
        pltpu.sync_copy(sv, us.at[dsc16], add=True)

    issue_sg(jnp.int32(0), bufs_a)
    issue_sg(jnp.int32(1), bufs_b)
    issue_gathers(jnp.int32(0), bufs_a)

    def cbody(u, carry):
        issue_gathers(2 * u + 1, bufs_b)
        drain(bufs_a)
        issue_sg(2 * u + 2, bufs_a)
        work(2 * u, bufs_a)
        issue_gathers(2 * u + 2, bufs_a)
        drain(bufs_b)
        issue_sg(2 * u + 3, bufs_b)
        work(2 * u + 1, bufs_b)
        return carry

    lax.fori_loop(0, CPC // 2, cbody, 0)
    drain(bufs_a)
    wait_sg(bufs_b)
    plsc.subcore_barrier()
    pltpu.sync_copy(uev.at[pl.ds(rb, nps)], uev_hbm.at[cid, pl.ds(rb, nps)])
    pltpu.sync_copy(us.at[pl.ds(rb2, nps2)], us_hbm.at[cid, pl.ds(rb2, nps2)])


def _sc_edge(xq, xk, xv, rad, srcx, dst, zer, zer2):
    dbl = [
        pltpu.VMEM((CB,), jnp.int32),
        pltpu.VMEM((CB,), jnp.int32),
        pltpu.VMEM((CB,), jnp.int32),
        pltpu.VMEM((CB,), jnp.int32),
        pltpu.VMEM((CB, HD), jnp.float32),
        pltpu.VMEM((CB, HD), jnp.float32),
        pltpu.VMEM((CB, HD), jnp.float32),
        pltpu.VMEM((CB, HD), jnp.float32),
        pltpu.VMEM((CB, HD), jnp.float32),
        pltpu.VMEM((CB, 16), jnp.float32),
    ]
    f = functools.partial(
        pl.kernel,
        out_type=(jax.ShapeDtypeStruct((NC, NPAD, HD), jnp.float32),
                  jax.ShapeDtypeStruct((NC, SR16, 16), jnp.float32)),
        mesh=_mesh(),
        compiler_params=_SC_PARAMS,
        scratch_types=(
            [pltpu.VMEM((EPC + LANES,), jnp.int32)]
            + dbl + dbl
            + [pltpu.VMEM_SHARED((NPAD, HD), jnp.float32),
               pltpu.VMEM_SHARED((SR16, 16), jnp.float32),
               pltpu.SemaphoreType.DMA,
               pltpu.SemaphoreType.DMA,
               pltpu.SemaphoreType.DMA,
               pltpu.SemaphoreType.DMA]),
    )(_edge_body)
    return f(xq, xk, xv, rad, srcx, dst, zer, zer2)


NB = 1000
EB = 2000


def _proj_body(x_ref, wq_ref, wk_ref, wv_ref, ws_ref,
               xq_ref, xk_ref, xv_ref, xs_ref):
    x = x_ref[...]
    xq = jnp.dot(x, wq_ref[...], precision=_HI)
    xk = jnp.dot(x, wk_ref[...], precision=_HI)
    xv = jnp.dot(x, wv_ref[...], precision=_HI)
    xq_ref[0] = xq[:, :HD]
    xq_ref[1] = xq[:, HD:]
    xk_ref[0] = xk[:, :HD]
    xk_ref[1] = xk[:, HD:]
    xv_ref[0] = xv[:, :HD]
    xv_ref[1] = xv[:, HD:]
    xs_ref[...] = jnp.dot(x, ws_ref[...], precision=_HI)


def _tc_proj(x, wq, wk, wv, ws):
    wspec = pl.BlockSpec((D, D), lambda i: (0, 0))
    nspec = pl.BlockSpec((NB, D), lambda i: (i, 0))
    hspec = pl.BlockSpec((NC, NB, HD), lambda i: (0, i, 0))
    hshape = jax.ShapeDtypeStruct((NC, N, HD), jnp.float32)
    xq, xk, xv, xs = pl.pallas_call(
        _proj_body,
        grid=(N // NB,),
        in_specs=[nspec, wspec, wspec, wspec, wspec],
        out_specs=[hspec, hspec, hspec, nspec],
        out_shape=[hshape, hshape, hshape,
                   jax.ShapeDtypeStruct((N, D), jnp.float32)],
    )(x, wq, wk, wv, ws)
    return (xq.reshape(NC * N, HD), xk.reshape(NC * N, HD),
            xv.reshape(NC * N, HD), xs)


def _radial_body(d2_ref, ef_ref, wd0_ref, we0_ref, b10_ref, g0_ref, be0_ref,
                 r20_ref, b20_ref, wd1_ref, we1_ref, b11_ref, g1_ref,
                 be1_ref, r21_ref, b21_ref, rad0_ref, rad1_ref):
    dist = jnp.sqrt(d2_ref[...] + 1e-12)
    ef = ef_ref[...]
    params = ((wd0_ref, we0_ref, b10_ref, g0_ref, be0_ref, r20_ref, b20_ref,
               rad0_ref),
              (wd1_ref, we1_ref, b11_ref, g1_ref, be1_ref, r21_ref, b21_ref,
               rad1_ref))
    for wd, we, b1, g, be, r2, b2, out in params:
        h = dist * wd[...] + jnp.dot(ef, we[...], precision=_HI) + b1[...]
        mu = jnp.mean(h, axis=1, keepdims=True)
        hc = h - mu
        var = jnp.mean(hc * hc, axis=1, keepdims=True)
        hn = hc / jnp.sqrt(var + 1e-5) * g[...] + be[...]
        hr = jnp.maximum(hn, 0.0)
        rad = jnp.dot(hr, r2[...], precision=_HI) + b2[...]
        out[0] = rad[:, :HD]
        out[1] = rad[:, HD:]


def _tc_radial(d2c, ef, R1, b1, lng, lnb, R2, b2):
    espec = pl.BlockSpec((EB, DE), lambda i: (i, 0))
    dspec = pl.BlockSpec((EB, 1), lambda i: (i, 0))
    ospec = pl.BlockSpec((NC, EB, HD), lambda i: (0, i, 0))

    def w(shape):
        return pl.BlockSpec(shape, lambda i: (0, 0))

    args = [d2c, ef]
    in_specs = [dspec, espec]
    for l in range(2):
        args += [R1[l, 0:1, :], R1[l, 1:, :], b1[l:l + 1, :],
                 lng[l:l + 1, :], lnb[l:l + 1, :], R2[l], b2[l:l + 1, :]]
        in_specs += [w((1, RH)), w((DE, RH)), w((1, RH)), w((1, RH)),
                     w((1, RH)), w((RH, D)), w((1, D))]
    rad0, rad1 = pl.pallas_call(
        _radial_body,
        grid=(E // EB,),
        in_specs=in_specs,
        out_specs=[ospec, ospec],
        out_shape=[jax.ShapeDtypeStruct((NC, E, HD), jnp.float32)] * 2,
    )(*args)
    return rad0.reshape(NC * E, HD), rad1.reshape(NC * E, HD)


def _combine(uev, sv, xs):
    agg_halves = []
    for c in range(NC):
        s = sv[c]
        agg_halves.append(uev[c] / (jnp.broadcast_to(s, (NB, HD)) + 1e-9))
    agg = jnp.concatenate(agg_halves, axis=1)
    return agg + xs


def _norm_se3(out, g, b):
    nrm = jnp.abs(out)
    phase = out / (nrm + 1e-8)
    return jnp.maximum(nrm * g + b, 0.0) * phase


def _combine_mid_body(u_ref, s_ref, xs_ref, g_ref, b_ref, xn_ref):
    out = _combine(u_ref[...], s_ref[...], xs_ref[...])
    xn_ref[...] = _norm_se3(out, g_ref[...], b_ref[...])


def _combine_final_body(u_ref, s_ref, xs_ref, g_ref, b_ref, wout_ref, y_ref):
    out = _combine(u_ref[...], s_ref[...], xs_ref[...])
    xn = _norm_se3(out, g_ref[...], b_ref[...])
    y_ref[...] = jnp.dot(xn, wout_ref[...], precision=_HI)


def _tc_combine(uev, sv, xs, g, b, wout=None):
    uspec = pl.BlockSpec((NC, NB, HD), lambda i: (0, i, 0))
    sspec = pl.BlockSpec((NC, NB, 1), lambda i: (0, i, 0))
    nspec = pl.BlockSpec((NB, D), lambda i: (i, 0))
    gspec = pl.BlockSpec((1, D), lambda i: (0, 0))
    out_shape = jax.ShapeDtypeStruct((N, D), jnp.float32)
    if wout is None:
        return pl.pallas_call(
            _combine_mid_body,
            grid=(N // NB,),
            in_specs=[uspec, sspec, nspec, gspec, gspec],
            out_specs=nspec,
            out_shape=out_shape,
        )(uev, sv, xs, g, b)
    return pl.pallas_call(
        _combine_final_body,
        grid=(N // NB,),
        in_specs=[uspec, sspec, nspec, gspec, gspec,
                  pl.BlockSpec((D, D), lambda i: (0, 0))],
        out_specs=nspec,
        out_shape=out_shape,
    )(uev, sv, xs, g, b, wout)


def kernel(node_feats, edge_feats, pos, edge_index, Wq, Wk, Wv, Wskip,
           R1, b1, lng, lnb, R2, b2, ng, nb, Wout):
    x = node_feats[..., 0]
    ef = edge_feats[..., 0]
    src = edge_index[0]
    dst = edge_index[1]
    px = jnp.asarray(pos[:, 0])
    py = jnp.asarray(pos[:, 1])
    pz = jnp.asarray(pos[:, 2])

    d2 = _sc_dist(px, py, pz, src, dst)
    rad0, rad1 = _tc_radial(d2.reshape(E, 1), ef, R1, b1, lng, lnb, R2, b2)
    zer = jnp.zeros((NPAD, HD), jnp.float32)
    zer2 = jnp.zeros((SR16, 16), jnp.float32)

    xq0, xk0, xv0, xs0 = _tc_proj(x, Wq[0], Wk[0], Wv[0], Wskip[0])
    srcx = jnp.stack([src, src + N])
    uev0, us0 = _sc_edge(xq0, xk0, xv0, rad0, srcx, dst, zer, zer2)
    x1 = _tc_combine(uev0, us0.reshape(NC, NPAD, 1), xs0,
                     ng[0:1, :], nb[0:1, :])

    xq1, xk1, xv1, xs1 = _tc_proj(x1, Wq[1], Wk[1], Wv[1], Wskip[1])
    uev1, us1 = _sc_edge(xq1, xk1, xv1, rad1, srcx, dst, zer, zer2)
    y = _tc_combine(uev1, us1.reshape(NC, NPAD, 1), xs1,
                    ng[1:2, :], nb[1:2, :], Wout)
    return y[..., None]


# speedup vs baseline: 28.5163x; 1.4505x over previous
import functools

import jax
import jax.numpy as jnp
from jax import lax
from jax.experimental import pallas as pl
from jax.experimental.pallas import tpu as pltpu
from jax.experimental.pallas import tpu_sc as plsc

NC = 2
NS = 16
LANES = 16

N = 10000
E = 320000
D = 128
DE = 16
H = 2
HD = D // H
RH = 32

NPAD = 10240
CB = 80
EPC = E // NS
CPC = EPC // CB
UW = 80

EPW32 = E // 32

_HI = jax.lax.Precision.HIGHEST


def _dot3(a, b):
    ah = a.astype(jnp.bfloat16).astype(jnp.float32)
    al = a - ah
    bh = b.astype(jnp.bfloat16).astype(jnp.float32)
    bl = b - bh
    return (jnp.dot(ah, bh, preferred_element_type=jnp.float32)
            + jnp.dot(ah, bl, preferred_element_type=jnp.float32)
            + jnp.dot(al, bh, preferred_element_type=jnp.float32))


def _mesh():
    return plsc.VectorSubcoreMesh(core_axis_name="c", subcore_axis_name="s")


_SC_PARAMS = pltpu.CompilerParams(needs_layout_passes=False,
                                  use_tc_tiling_on_sc=False)


def _dist_body(px_hbm, py_hbm, pz_hbm, src_hbm, dst_hbm, d2_hbm,
               px_v, py_v, pz_v, src_v, dst_v, d2_v):
    wid = lax.axis_index("s") * NC + lax.axis_index("c")
    base = pl.multiple_of(wid * EPW32, 8)
    pltpu.sync_copy(px_hbm, px_v)
    pltpu.sync_copy(py_hbm, py_v)
    pltpu.sync_copy(pz_hbm, pz_v)
    pltpu.sync_copy(src_hbm.at[pl.ds(base, EPW32)], src_v)
    pltpu.sync_copy(dst_hbm.at[pl.ds(base, EPW32)], dst_v)

    def body(g, carry):
        si = src_v[pl.ds(g * LANES, LANES)]
        di = dst_v[pl.ds(g * LANES, LANES)]
        dx = plsc.load_gather(px_v, [di]) - plsc.load_gather(px_v, [si])
        dy = plsc.load_gather(py_v, [di]) - plsc.load_gather(py_v, [si])
        dz = plsc.load_gather(pz_v, [di]) - plsc.load_gather(pz_v, [si])
        d2_v[pl.ds(g * LANES, LANES)] = dx * dx + dy * dy + dz * dz
        return carry

    lax.fori_loop(0, EPW32 // LANES, body, 0)
    pltpu.sync_copy(d2_v, d2_hbm.at[pl.ds(base, EPW32)])


def _sc_dist(px, py, pz, src, dst):
    f = functools.partial(
        pl.kernel,
        out_type=jax.ShapeDtypeStruct((E,), jnp.float32),
        mesh=_mesh(),
        compiler_params=_SC_PARAMS,
        scratch_types=[
            pltpu.VMEM((N,), jnp.float32),
            pltpu.VMEM((N,), jnp.float32),
            pltpu.VMEM((N,), jnp.float32),
            pltpu.VMEM((EPW32,), jnp.int32),
            pltpu.VMEM((EPW32,), jnp.int32),
            pltpu.VMEM((EPW32,), jnp.float32),
        ],
    )(_dist_body)
    return f(px, py, pz, src, dst)


def _edge_body(xq_hbm, xk_hbm, xv_hbm, rad_hbm, srcx_hbm, dst_hbm,
               zer_hbm, u_hbm,
               dstall,
               sga, dga, dsa, qva, kva, vva, rva, wva,
               sgb, dgb, dsb, qvb, kvb, vvb, rvb, wvb,
               uac, sema, semb, semsga, semsgb):
    cid = lax.axis_index("c")
    sid = lax.axis_index("s")

    rb = sid * (NPAD // NS)
    nps = NPAD // NS
    pltpu.sync_copy(zer_hbm.at[pl.ds(rb, nps)], uac.at[pl.ds(rb, nps)])

    ebase = pl.multiple_of(sid * EPC, 8)
    pltpu.sync_copy(dst_hbm.at[pl.ds(ebase, EPC)], dstall)
    plsc.subcore_barrier()

    lane = lax.broadcasted_iota(jnp.int32, (LANES,), 0)
    noff = cid * N

    bufs_a = (sga, dga, dsa, qva, kva, vva, rva, wva, sema, semsga)
    bufs_b = (sgb, dgb, dsb, qvb, kvb, vvb, rvb, wvb, semb, semsgb)

    def issue_sg(t, bufs):
        sg, sem = bufs[0], bufs[9]
        t = jnp.minimum(t, CPC - 1)
        base = pl.multiple_of(sid * EPC, 8) + t * CB
        pltpu.async_copy(srcx_hbm.at[cid, pl.ds(base, CB)], sg, sem)

    def wait_sg(bufs):
        sg, sem = bufs[0], bufs[9]
        pltpu.make_async_copy(srcx_hbm.at[cid, pl.ds(0, CB)], sg, sem).wait()

    def issue_gathers(t, bufs):
        sg, dg, dsc, qv, kv, vv, rv, wv, sem = bufs[:9]
        t = jnp.minimum(t, CPC - 1)
        off = t * CB
        for g in range(CB // LANES):
            dsg = pl.ds(g * LANES, LANES)
            d16 = dstall[pl.ds(off + g * LANES, LANES)]
            dg[dsg] = d16 + noff
            dsc[dsg] = d16
        wait_sg(bufs)
        pltpu.async_copy(xq_hbm.at[dg], qv, sem)
        pltpu.async_copy(xk_hbm.at[sg], kv, sem)
        pltpu.async_copy(xv_hbm.at[sg], vv, sem)
        rbase = pl.multiple_of(cid * E + sid * EPC, 8) + t * CB
        pltpu.async_copy(rad_hbm.at[pl.ds(rbase, CB)], rv, sem)

    def drain(bufs):
        sg, dg, dsc, qv, kv, vv, rv, wv, sem = bufs[:9]
        pltpu.make_async_copy(xq_hbm.at[dg], qv, sem).wait()
        pltpu.make_async_copy(xk_hbm.at[sg], kv, sem).wait()
        pltpu.make_async_copy(xv_hbm.at[sg], vv, sem).wait()
        pltpu.make_async_copy(rad_hbm.at[pl.ds(0, CB)], rv, sem).wait()

    def compute(bufs):
        sg, dg, dsc, qv, kv, vv, rv, wv = bufs[:8]

        def ebody(i, carry):
            h = jnp.zeros((LANES,), jnp.float32)
            vrs = []
            for j in range(4):
                dsj = pl.ds(j * LANES, LANES)
                rj = rv[i, dsj]
                h = h + qv[i, dsj] * (kv[i, dsj] * rj)
                vrs.append(vv[i, dsj] * rj)
            l = jnp.minimum(jnp.sum(h) * 0.125, 75.0)
            ev = jnp.exp(jnp.full((LANES,), l, jnp.float32))
            for j in range(4):
                wv[i, pl.ds(j * LANES, LANES)] = vrs[j] * ev
            wv[i, pl.ds(HD, LANES)] = jnp.where(lane == 0, ev, 0.0)
            return carry

        lax.fori_loop(0, CB, ebody, 0, unroll=2)
        pltpu.sync_copy(wv, uac.at[dsc], add=True)

    issue_sg(jnp.int32(0), bufs_a)
    issue_sg(jnp.int32(1), bufs_b)
    issue_gathers(jnp.int32(0), bufs_a)

    def cbody(u, carry):
        issue_gathers(2 * u + 1, bufs_b)
        drain(bufs_a)
        issue_sg(2 * u + 2, bufs_a)
        compute(bufs_a)
        issue_gathers(2 * u + 2, bufs_a)
        drain(bufs_b)
        issue_sg(2 * u + 3, bufs_b)
        compute(bufs_b)
        return carry

    lax.fori_loop(0, CPC // 2, cbody, 0)
    drain(bufs_a)
    wait_sg(bufs_b)
    plsc.subcore_barrier()
    pltpu.sync_copy(uac.at[pl.ds(rb, nps)], u_hbm.at[cid, pl.ds(rb, nps)])


def _sc_edge(xq, xk, xv, rad, srcx, dst, zer):
    dbl = [
        pltpu.VMEM((CB,), jnp.int32),
        pltpu.VMEM((CB,), jnp.int32),
        pltpu.VMEM((CB,), jnp.int32),
        pltpu.VMEM((CB, HD), jnp.float32),
        pltpu.VMEM((CB, HD), jnp.float32),
        pltpu.VMEM((CB, HD), jnp.float32),
        pltpu.VMEM((CB, HD), jnp.float32),
        pltpu.VMEM((CB, UW), jnp.float32),
    ]
    f = functools.partial(
        pl.kernel,
        out_type=jax.ShapeDtypeStruct((NC, NPAD, UW), jnp.float32),
        mesh=_mesh(),
        compiler_params=_SC_PARAMS,
        scratch_types=(
            [pltpu.VMEM((EPC,), jnp.int32)]
            + dbl + dbl
            + [pltpu.VMEM_SHARED((NPAD, UW), jnp.float32),
               pltpu.SemaphoreType.DMA,
               pltpu.SemaphoreType.DMA,
               pltpu.SemaphoreType.DMA,
               pltpu.SemaphoreType.DMA]),
    )(_edge_body)
    return f(xq, xk, xv, rad, srcx, dst, zer)


NB = 1000
EB = 4000


def _proj_body(x_ref, wq_ref, wk_ref, wv_ref, ws_ref,
               xq_ref, xk_ref, xv_ref, xs_ref):
    x = x_ref[...]
    xq = jnp.dot(x, wq_ref[...], precision=_HI)
    xk = jnp.dot(x, wk_ref[...], precision=_HI)
    xv = jnp.dot(x, wv_ref[...], precision=_HI)
    xq_ref[0] = xq[:, :HD]
    xq_ref[1] = xq[:, HD:]
    xk_ref[0] = xk[:, :HD]
    xk_ref[1] = xk[:, HD:]
    xv_ref[0] = xv[:, :HD]
    xv_ref[1] = xv[:, HD:]
    xs_ref[...] = jnp.dot(x, ws_ref[...], precision=_HI)


def _tc_proj(x, wq, wk, wv, ws):
    wspec = pl.BlockSpec((D, D), lambda i: (0, 0))
    nspec = pl.BlockSpec((NB, D), lambda i: (i, 0))
    hspec = pl.BlockSpec((NC, NB, HD), lambda i: (0, i, 0))
    hshape = jax.ShapeDtypeStruct((NC, N, HD), jnp.float32)
    xq, xk, xv, xs = pl.pallas_call(
        _proj_body,
        grid=(N // NB,),
        in_specs=[nspec, wspec, wspec, wspec, wspec],
        out_specs=[hspec, hspec, hspec, nspec],
        out_shape=[hshape, hshape, hshape,
                   jax.ShapeDtypeStruct((N, D), jnp.float32)],
    )(x, wq, wk, wv, ws)
    return (xq.reshape(NC * N, HD), xk.reshape(NC * N, HD),
            xv.reshape(NC * N, HD), xs)


def _radial_body(d2_ref, ef_ref, wd_ref, we_ref, b1_ref, g_ref, be_ref,
                 r2_ref, b2_ref, rad_ref):
    dist = jnp.sqrt(d2_ref[...] + 1e-12)
    ef = ef_ref[...]
    h = dist * wd_ref[...] + _dot3(ef, we_ref[...]) + b1_ref[...]
    mu = jnp.mean(h, axis=1, keepdims=True)
    hc = h - mu
    var = jnp.mean(hc * hc, axis=1, keepdims=True)
    hn = hc / jnp.sqrt(var + 1e-5) * g_ref[...] + be_ref[...]
    hr = jnp.maximum(hn, 0.0)
    rad = _dot3(hr, r2_ref[...]) + b2_ref[...]
    rad_ref[0] = rad[:, :HD]
    rad_ref[1] = rad[:, HD:]


def _tc_radial(l, d2c, ef, R1, b1, lng, lnb, R2, b2):
    espec = pl.BlockSpec((EB, DE), lambda i: (i, 0))
    dspec = pl.BlockSpec((EB, 1), lambda i: (i, 0))
    ospec = pl.BlockSpec((NC, EB, HD), lambda i: (0, i, 0))

    def w(shape):
        return pl.BlockSpec(shape, lambda i: (0, 0))

    args = [d2c, ef, R1[l, 0:1, :], R1[l, 1:, :], b1[l:l + 1, :],
            lng[l:l + 1, :], lnb[l:l + 1, :], R2[l], b2[l:l + 1, :]]
    in_specs = [dspec, espec, w((1, RH)), w((DE, RH)), w((1, RH)),
                w((1, RH)), w((1, RH)), w((RH, D)), w((1, D))]
    rad = pl.pallas_call(
        _radial_body,
        grid=(E // EB,),
        in_specs=in_specs,
        out_specs=ospec,
        out_shape=jax.ShapeDtypeStruct((NC, E, HD), jnp.float32),
    )(*args)
    return rad.reshape(NC * E, HD)


def _combine(u, xs):
    agg_halves = []
    for c in range(NC):
        sc = u[c][:, HD:HD + 1]
        agg_halves.append(u[c][:, :HD]
                          / (jnp.broadcast_to(sc, (NB, HD)) + 1e-9))
    agg = jnp.concatenate(agg_halves, axis=1)
    return agg + xs


def _norm_se3(out, g, b):
    nrm = jnp.abs(out)
    phase = out / (nrm + 1e-8)
    return jnp.maximum(nrm * g + b, 0.0) * phase


def _combine_proj_body(u_ref, xs_ref, g_ref, b_ref,
                       wq_ref, wk_ref, wv_ref, ws_ref,
                       xq_ref, xk_ref, xv_ref, xso_ref):
    out = _combine(u_ref[...], xs_ref[...])
    x = _norm_se3(out, g_ref[...], b_ref[...])
    xq = jnp.dot(x, wq_ref[...], precision=_HI)
    xk = jnp.dot(x, wk_ref[...], precision=_HI)
    xv = jnp.dot(x, wv_ref[...], precision=_HI)
    xq_ref[0] = xq[:, :HD]
    xq_ref[1] = xq[:, HD:]
    xk_ref[0] = xk[:, :HD]
    xk_ref[1] = xk[:, HD:]
    xv_ref[0] = xv[:, :HD]
    xv_ref[1] = xv[:, HD:]
    xso_ref[...] = jnp.dot(x, ws_ref[...], precision=_HI)


def _tc_combine_proj(u, xs, g, b, wq, wk, wv, ws):
    uspec = pl.BlockSpec((NC, NB, UW), lambda i: (0, i, 0))
    nspec = pl.BlockSpec((NB, D), lambda i: (i, 0))
    gspec = pl.BlockSpec((1, D), lambda i: (0, 0))
    wspec = pl.BlockSpec((D, D), lambda i: (0, 0))
    hspec = pl.BlockSpec((NC, NB, HD), lambda i: (0, i, 0))
    hshape = jax.ShapeDtypeStruct((NC, N, HD), jnp.float32)
    xq, xk, xv, xso = pl.pallas_call(
        _combine_proj_body,
        grid=(N // NB,),
        in_specs=[uspec, nspec, gspec, gspec,
                  wspec, wspec, wspec, wspec],
        out_specs=[hspec, hspec, hspec, nspec],
        out_shape=[hshape, hshape, hshape,
                   jax.ShapeDtypeStruct((N, D), jnp.float32)],
    )(u, xs, g, b, wq, wk, wv, ws)
    return (xq.reshape(NC * N, HD), xk.reshape(NC * N, HD),
            xv.reshape(NC * N, HD), xso)


def _combine_final_body(u_ref, xs_ref, g_ref, b_ref, wout_ref, y_ref):
    out = _combine(u_ref[...], xs_ref[...])
    xn = _norm_se3(out, g_ref[...], b_ref[...])
    y_ref[...] = jnp.dot(xn, wout_ref[...], precision=_HI)


def _tc_combine_final(u, xs, g, b, wout):
    uspec = pl.BlockSpec((NC, NB, UW), lambda i: (0, i, 0))
    nspec = pl.BlockSpec((NB, D), lambda i: (i, 0))
    gspec = pl.BlockSpec((1, D), lambda i: (0, 0))
    out_shape = jax.ShapeDtypeStruct((N, D), jnp.float32)
    return pl.pallas_call(
        _combine_final_body,
        grid=(N // NB,),
        in_specs=[uspec, nspec, gspec, gspec,
                  pl.BlockSpec((D, D), lambda i: (0, 0))],
        out_specs=nspec,
        out_shape=out_shape,
    )(u, xs, g, b, wout)


def kernel(node_feats, edge_feats, pos, edge_index, Wq, Wk, Wv, Wskip,
           R1, b1, lng, lnb, R2, b2, ng, nb, Wout):
    x = node_feats[..., 0]
    ef = edge_feats[..., 0]
    src = edge_index[0]
    dst = edge_index[1]
    px = jnp.asarray(pos[:, 0])
    py = jnp.asarray(pos[:, 1])
    pz = jnp.asarray(pos[:, 2])

    d2 = _sc_dist(px, py, pz, src, dst)
    d2c = d2.reshape(E, 1)
    rad0 = _tc_radial(0, d2c, ef, R1, b1, lng, lnb, R2, b2)
    zer = jnp.zeros((NPAD, UW), jnp.float32)

    xq0, xk0, xv0, xs0 = _tc_proj(x, Wq[0], Wk[0], Wv[0], Wskip[0])
    srcx = jnp.stack([src, src + N])
    u0 = _sc_edge(xq0, xk0, xv0, rad0, srcx, dst, zer)
    rad1 = _tc_radial(1, d2c, ef, R1, b1, lng, lnb, R2, b2)
    xq1, xk1, xv1, xs1 = _tc_combine_proj(
        u0, xs0, ng[0:1, :], nb[0:1, :], Wq[1], Wk[1], Wv[1], Wskip[1])
    u1 = _sc_edge(xq1, xk1, xv1, rad1, srcx, dst, zer)
    y = _tc_combine_final(u1, xs1, ng[1:2, :], nb[1:2, :], Wout)
    return y[..., None]
